# baseline (speedup 1.0000x reference)
import functools

import jax
import jax.numpy as jnp
from jax import lax
from jax.experimental import pallas as pl
from jax.experimental.pallas import tpu as pltpu
from jax.experimental.pallas import tpu_sc as plsc

D = 64
DP = 128
SCALE = 8.0
NC = 2
NS = 16
NW = NC * NS
NBUF = 4
CH = 128


@functools.lru_cache(maxsize=None)
def _make(S: int, L: int, V: int):
    LP = -(-L // 8) * 8
    SPW = S // NW
    G = SPW // 2
    W = 2 * LP
    assert S % (2 * NW) == 0 and W <= CH
    mesh = plsc.VectorSubcoreMesh(core_axis_name="c", subcore_axis_name="s")

    @functools.partial(
        pl.kernel,
        mesh=mesh,
        out_type=jax.ShapeDtypeStruct((S * LP, DP), jnp.float32),
        compiler_params=pltpu.CompilerParams(use_tc_tiling_on_sc=False),
        scratch_types=[
            pltpu.VMEM((G, CH), jnp.int32),
            *[pltpu.VMEM((CH, DP), jnp.float32) for _ in range(NBUF)],
            *[pltpu.SemaphoreType.DMA for _ in range(2 * NBUF)],
        ],
    )
    def emb(x_hbm, lut_hbm, out_hbm, idx_v, r0, r1, r2, r3,
            g0, g1, g2, g3, s0, s1, s2, s3):
        bufs = (r0, r1, r2, r3)
        gsem = (g0, g1, g2, g3)
        ssem = (s0, s1, s2, s3)
        wid = lax.axis_index("s") * NC + lax.axis_index("c")
        base = wid * SPW * LP

        pltpu.sync_copy(x_hbm.at[wid], idx_v)

        def start_gather(g, b):
            pltpu.async_copy(lut_hbm.at[idx_v.at[g]], bufs[b], gsem[b])

        def wait_gather(g, b):
            pltpu.make_async_copy(lut_hbm.at[idx_v.at[g]], bufs[b],
                                  gsem[b]).wait()

        def start_scatter(g, b):
            pltpu.async_copy(bufs[b].at[pl.ds(0, W)],
                             out_hbm.at[pl.ds(base + g * W, W)], ssem[b])

        def wait_scatter(g, b):
            pltpu.make_async_copy(bufs[b].at[pl.ds(0, W)],
                                  out_hbm.at[pl.ds(base + g * W, W)],
                                  ssem[b]).wait()

        start_gather(0, 0)
        start_gather(1, 1)

        def body(i, carry):
            for b in range(NBUF):
                g = i * NBUF + b
                bn = (b + 2) % NBUF
                pl.when(g >= 2)(lambda: wait_scatter(g - 2, bn))
                pl.when(g + 2 < G)(lambda: start_gather(g + 2, bn))
                wait_gather(g, b)
                start_scatter(g, b)
            return carry

        lax.fori_loop(0, G // NBUF, body, 0)

        wait_scatter(G - 2, (G - 2) % NBUF)
        wait_scatter(G - 1, (G - 1) % NBUF)

    return emb


def kernel(x, lut):
    S, L = x.shape
    V = lut.shape[0]
    LP = -(-L // 8) * 8
    lut8 = lut * SCALE
    lutp = jnp.concatenate([lut8, lut8], axis=1)
    x3 = jnp.pad(x.astype(jnp.int32).reshape(NW, S // NW, L),
                 ((0, 0), (0, 0), (0, LP - L)))
    x3 = x3.reshape(NW, S // NW // 2, 2 * LP)
    x3 = jnp.pad(x3, ((0, 0), (0, 0), (0, CH - 2 * LP)))
    out = _make(S, L, V)(x3, lutp)
    return out.reshape(S, LP, DP)[:, :L, :D]


# speedup vs baseline: 6.3188x; 6.3188x over previous
import functools

import jax
import jax.numpy as jnp
from jax import lax
from jax.experimental import pallas as pl
from jax.experimental.pallas import tpu as pltpu
from jax.experimental.pallas import tpu_sc as plsc

D = 64
DP = 128
SCALE = 8.0
NC = 2
NS = 16
NW = NC * NS
NBUF = 4
CH = 128


@functools.lru_cache(maxsize=None)
def _make(S: int, L: int, V: int):
    LP = -(-L // 8) * 8
    SPW = S // NW
    G = SPW // 2
    W = 2 * LP
    assert S % (2 * NW) == 0 and W <= CH
    mesh = plsc.VectorSubcoreMesh(core_axis_name="c", subcore_axis_name="s")

    @functools.partial(
        pl.kernel,
        mesh=mesh,
        out_type=jax.ShapeDtypeStruct((S * LP, DP), jnp.float32),
        compiler_params=pltpu.CompilerParams(use_tc_tiling_on_sc=False),
        scratch_types=[
            pltpu.VMEM((G, CH), jnp.int32),
            *[pltpu.VMEM((CH, DP), jnp.float32) for _ in range(NBUF)],
            *[pltpu.SemaphoreType.DMA for _ in range(2 * NBUF)],
        ],
    )
    def emb(x_hbm, lut_hbm, out_hbm, idx_v, r0, r1, r2, r3,
            g0, g1, g2, g3, s0, s1, s2, s3):
        bufs = (r0, r1, r2, r3)
        gsem = (g0, g1, g2, g3)
        ssem = (s0, s1, s2, s3)
        wid = lax.axis_index("s") * NC + lax.axis_index("c")
        base = wid * SPW * LP

        pltpu.sync_copy(x_hbm.at[wid], idx_v)

        def start_gather(g, b):
            pltpu.async_copy(lut_hbm.at[idx_v.at[g]], bufs[b], gsem[b])

        def wait_gather(g, b):
            pltpu.make_async_copy(lut_hbm.at[idx_v.at[g]], bufs[b],
                                  gsem[b]).wait()

        def start_scatter(g, b):
            pltpu.async_copy(bufs[b].at[pl.ds(0, W)],
                             out_hbm.at[pl.ds(base + g * W, W)], ssem[b])

        def wait_scatter(g, b):
            pltpu.make_async_copy(bufs[b].at[pl.ds(0, W)],
                                  out_hbm.at[pl.ds(base + g * W, W)],
                                  ssem[b]).wait()

        start_gather(0, 0)
        start_gather(1, 1)

        def body(i, carry):
            for b in range(NBUF):
                g = i * NBUF + b
                bn = (b + 2) % NBUF
                pl.when(g >= 2)(lambda: wait_scatter(g - 2, bn))
                pl.when(g + 2 < G)(lambda: start_gather(g + 2, bn))
                wait_gather(g, b)
                start_scatter(g, b)
            return carry

        lax.fori_loop(0, G // NBUF, body, 0)

        wait_scatter(G - 2, (G - 2) % NBUF)
        wait_scatter(G - 1, (G - 1) % NBUF)

    return emb


def kernel(x, lut):
    S, L = x.shape
    V = lut.shape[0]
    LP = -(-L // 8) * 8
    lut8 = lut * SCALE
    lutp = jnp.concatenate([lut8, lut8], axis=1)
    x3 = jnp.pad(x.astype(jnp.int32).reshape(NW, S // NW, L),
                 ((0, 0), (0, 0), (0, LP - L)), mode="edge")
    x3 = x3.reshape(NW, S // NW // 2, 2 * LP)
    x3 = jnp.pad(x3, ((0, 0), (0, 0), (0, CH - 2 * LP)), mode="edge")
    out = _make(S, L, V)(x3, lutp)
    return out.reshape(S, LP, DP)[:, :L, :D]


# speedup vs baseline: 8.5094x; 1.3467x over previous
import functools

import jax
import jax.numpy as jnp
from jax import lax
from jax.experimental import pallas as pl
from jax.experimental.pallas import tpu as pltpu
from jax.experimental.pallas import tpu_sc as plsc

D = 64
DP = 128
SCALE = 8.0
NC = 2
NS = 16
NW = NC * NS
NBUF = 4
CH = 128


@functools.lru_cache(maxsize=None)
def _make(S: int, L: int, V: int):
    LP = -(-L // 8) * 8
    SPW = S // NW
    G = SPW // 2
    W = 2 * LP
    assert S % (2 * NW) == 0 and W <= CH
    mesh = plsc.VectorSubcoreMesh(core_axis_name="c", subcore_axis_name="s")

    @functools.partial(
        pl.kernel,
        mesh=mesh,
        out_type=jax.ShapeDtypeStruct((S * LP, DP), jnp.float32),
        compiler_params=pltpu.CompilerParams(use_tc_tiling_on_sc=False),
        scratch_types=[
            pltpu.VMEM((G, CH), jnp.int32),
            *[pltpu.VMEM((CH, DP), jnp.float32) for _ in range(NBUF)],
            *[pltpu.SemaphoreType.DMA for _ in range(2 * NBUF)],
        ],
    )
    def emb(x_hbm, lut_hbm, out_hbm, idx_v, r0, r1, r2, r3,
            g0, g1, g2, g3, s0, s1, s2, s3):
        bufs = (r0, r1, r2, r3)
        gsem = (g0, g1, g2, g3)
        ssem = (s0, s1, s2, s3)
        wid = lax.axis_index("s") * NC + lax.axis_index("c")
        base = wid * SPW * LP

        pltpu.sync_copy(x_hbm.at[wid], idx_v)

        def start_gather(g, b):
            pltpu.async_copy(lut_hbm.at[idx_v.at[g]], bufs[b], gsem[b])

        def wait_gather(g, b):
            pltpu.make_async_copy(lut_hbm.at[idx_v.at[g]], bufs[b],
                                  gsem[b]).wait()

        def start_scatter(g, b):
            pltpu.async_copy(bufs[b].at[pl.ds(0, W)],
                             out_hbm.at[pl.ds(base + g * W, W)], ssem[b])

        def wait_scatter(g, b):
            pltpu.make_async_copy(bufs[b].at[pl.ds(0, W)],
                                  out_hbm.at[pl.ds(base + g * W, W)],
                                  ssem[b]).wait()

        start_gather(0, 0)
        start_gather(1, 1)

        def scale(b):
            buf = bufs[b]

            def row(r, carry):
                for c in range(D // 16):
                    buf[r, pl.ds(c * 16, 16)] = (
                        buf[r, pl.ds(c * 16, 16)] * SCALE)
                return carry

            lax.fori_loop(0, W, row, 0)

        def body(i, carry):
            for b in range(NBUF):
                g = i * NBUF + b
                bn = (b + 2) % NBUF
                pl.when(g >= 2)(lambda: wait_scatter(g - 2, bn))
                pl.when(g + 2 < G)(lambda: start_gather(g + 2, bn))
                wait_gather(g, b)
                scale(b)
                start_scatter(g, b)
            return carry

        lax.fori_loop(0, G // NBUF, body, 0)

        wait_scatter(G - 2, (G - 2) % NBUF)
        wait_scatter(G - 1, (G - 1) % NBUF)

    return emb


def kernel(x, lut):
    S, L = x.shape
    V = lut.shape[0]
    LP = -(-L // 8) * 8
    lutp = jnp.pad(lut, ((0, 0), (0, DP - lut.shape[1])))
    x3 = jnp.pad(x.astype(jnp.int32).reshape(NW, S // NW, L),
                 ((0, 0), (0, 0), (0, LP - L)), mode="edge")
    x3 = x3.reshape(NW, S // NW // 2, 2 * LP)
    x3 = jnp.pad(x3, ((0, 0), (0, 0), (0, CH - 2 * LP)), mode="edge")
    out = _make(S, L, V)(x3, lutp)
    return out.reshape(S, LP, DP)[:, :L, :D]


# speedup vs baseline: 8.5464x; 1.0043x over previous
import functools

import jax
import jax.numpy as jnp
from jax import lax
from jax.experimental import pallas as pl
from jax.experimental.pallas import tpu as pltpu
from jax.experimental.pallas import tpu_sc as plsc

D = 64
DP = 128
SCALE = 8.0
NC = 2
NS = 16
NW = NC * NS
NBUF = 4
CH = 128


@functools.lru_cache(maxsize=None)
def _make(S: int, L: int, V: int):
    LP = -(-L // 8) * 8
    SPW = S // NW
    G = SPW // 2
    W = 2 * LP
    assert S % (2 * NW) == 0 and W <= CH
    mesh = plsc.VectorSubcoreMesh(core_axis_name="c", subcore_axis_name="s")

    @functools.partial(
        pl.kernel,
        mesh=mesh,
        out_type=jax.ShapeDtypeStruct((S * LP, DP), jnp.float32),
        compiler_params=pltpu.CompilerParams(use_tc_tiling_on_sc=False),
        scratch_types=[
            pltpu.VMEM((G, CH), jnp.int32),
            *[pltpu.VMEM((CH, DP), jnp.float32) for _ in range(NBUF)],
            *[pltpu.SemaphoreType.DMA for _ in range(2 * NBUF)],
        ],
    )
    def emb(x_hbm, lut_hbm, out_hbm, idx_v, r0, r1, r2, r3,
            g0, g1, g2, g3, s0, s1, s2, s3):
        bufs = (r0, r1, r2, r3)
        gsem = (g0, g1, g2, g3)
        ssem = (s0, s1, s2, s3)
        wid = lax.axis_index("s") * NC + lax.axis_index("c")
        base = wid * SPW * LP

        pltpu.sync_copy(x_hbm.at[wid], idx_v)

        def start_gather(g, b):
            pltpu.async_copy(lut_hbm.at[idx_v.at[g]], bufs[b], gsem[b])

        def wait_gather(g, b):
            pltpu.make_async_copy(lut_hbm.at[idx_v.at[g]], bufs[b],
                                  gsem[b]).wait()

        def start_scatter(g, b):
            pltpu.async_copy(bufs[b].at[pl.ds(0, W)],
                             out_hbm.at[pl.ds(base + g * W, W)], ssem[b])

        def wait_scatter(g, b):
            pltpu.make_async_copy(bufs[b].at[pl.ds(0, W)],
                                  out_hbm.at[pl.ds(base + g * W, W)],
                                  ssem[b]).wait()

        start_gather(0, 0)
        start_gather(1, 1)

        def scale(b):
            buf = bufs[b]

            def row(r, carry):
                for c in range(D // 16):
                    buf[r, pl.ds(c * 16, 16)] = (
                        buf[r, pl.ds(c * 16, 16)] * SCALE)
                return carry

            lax.fori_loop(0, W, row, 0)

        def body(i, carry):
            for b in range(NBUF):
                g = i * NBUF + b
                bn = (b + 2) % NBUF
                pl.when(g >= 2)(lambda: wait_scatter(g - 2, bn))
                pl.when(g + 2 < G)(lambda: start_gather(g + 2, bn))
                wait_gather(g, b)
                scale(b)
                start_scatter(g, b)
            return carry

        lax.fori_loop(0, G // NBUF, body, 0)

        wait_scatter(G - 2, (G - 2) % NBUF)
        wait_scatter(G - 1, (G - 1) % NBUF)

    return emb


def kernel(x, lut):
    S, L = x.shape
    V = lut.shape[0]
    LP = -(-L // 8) * 8
    lutp = jnp.concatenate(
        [lut, jnp.zeros((V, DP - lut.shape[1]), lut.dtype)], axis=1)
    x3 = jnp.pad(x.astype(jnp.int32).reshape(NW, S // NW, L),
                 ((0, 0), (0, 0), (0, LP - L)), mode="edge")
    x3 = x3.reshape(NW, S // NW // 2, 2 * LP)
    x3 = jnp.pad(x3, ((0, 0), (0, 0), (0, CH - 2 * LP)), mode="edge")
    out = _make(S, L, V)(x3, lutp)
    return out.reshape(S, LP, DP)[:, :L, :D]


# speedup vs baseline: 8.6563x; 1.0129x over previous
import functools

import jax
import jax.numpy as jnp
from jax import lax
from jax.experimental import pallas as pl
from jax.experimental.pallas import tpu as pltpu
from jax.experimental.pallas import tpu_sc as plsc

D = 64
DP = 128
SCALE = 8.0
NC = 2
NS = 16
NW = NC * NS
NBUF = 4
CH = 128


@functools.lru_cache(maxsize=None)
def _make(S: int, L: int, V: int):
    LP = -(-L // 8) * 8
    SPW = S // NW
    G = SPW // 2
    W = 2 * LP
    assert S % (2 * NW) == 0 and W <= CH
    mesh = plsc.VectorSubcoreMesh(core_axis_name="c", subcore_axis_name="s")

    @functools.partial(
        pl.kernel,
        mesh=mesh,
        out_type=jax.ShapeDtypeStruct((S * LP, DP), jnp.float32),
        compiler_params=pltpu.CompilerParams(use_tc_tiling_on_sc=False),
        scratch_types=[
            pltpu.VMEM((G, CH), jnp.int32),
            *[pltpu.VMEM((CH, DP), jnp.float32) for _ in range(NBUF)],
            *[pltpu.SemaphoreType.DMA for _ in range(2 * NBUF)],
        ],
    )
    def emb(x_hbm, lut_hbm, out_hbm, idx_v, r0, r1, r2, r3,
            g0, g1, g2, g3, s0, s1, s2, s3):
        bufs = (r0, r1, r2, r3)
        gsem = (g0, g1, g2, g3)
        ssem = (s0, s1, s2, s3)
        wid = lax.axis_index("s") * NC + lax.axis_index("c")
        base = wid * SPW * LP

        pltpu.sync_copy(x_hbm.at[wid], idx_v)

        def start_gather(g, b):
            pltpu.async_copy(lut_hbm.at[idx_v.at[g]], bufs[b], gsem[b])

        def wait_gather(g, b):
            pltpu.make_async_copy(lut_hbm.at[idx_v.at[g]], bufs[b],
                                  gsem[b]).wait()

        def start_scatter(g, b):
            pltpu.async_copy(bufs[b].at[pl.ds(0, W)],
                             out_hbm.at[pl.ds(base + g * W, W)], ssem[b])

        def wait_scatter(g, b):
            pltpu.make_async_copy(bufs[b].at[pl.ds(0, W)],
                                  out_hbm.at[pl.ds(base + g * W, W)],
                                  ssem[b]).wait()

        start_gather(0, 0)
        start_gather(1, 1)

        def body(i, carry):
            for b in range(NBUF):
                g = i * NBUF + b
                bn = (b + 2) % NBUF
                pl.when(g >= 2)(lambda: wait_scatter(g - 2, bn))
                pl.when(g + 2 < G)(lambda: start_gather(g + 2, bn))
                wait_gather(g, b)
                start_scatter(g, b)
            return carry

        lax.fori_loop(0, G // NBUF, body, 0)

        wait_scatter(G - 2, (G - 2) % NBUF)
        wait_scatter(G - 1, (G - 1) % NBUF)

    return emb


def kernel(x, lut):
    S, L = x.shape
    V = lut.shape[0]
    LP = -(-L // 8) * 8
    lutp = jnp.pad(lut, ((0, 0), (0, DP - lut.shape[1]))) * SCALE
    x3 = jnp.pad(x.astype(jnp.int32).reshape(NW, S // NW, L),
                 ((0, 0), (0, 0), (0, LP - L)), mode="edge")
    x3 = x3.reshape(NW, S // NW // 2, 2 * LP)
    x3 = jnp.pad(x3, ((0, 0), (0, 0), (0, CH - 2 * LP)), mode="edge")
    out = _make(S, L, V)(x3, lutp)
    return out.reshape(S, LP, DP)[:, :L, :D]


# speedup vs baseline: 10.2577x; 1.1850x over previous
import functools

import jax
import jax.numpy as jnp
from jax import lax
from jax.experimental import pallas as pl
from jax.experimental.pallas import tpu as pltpu
from jax.experimental.pallas import tpu_sc as plsc

D = 64
DP = 128
SCALE = 8.0
NC = 2
NS = 16
NW = NC * NS
NBUF = 8
AHEAD = 6
CH = 128
NQ = 4


@functools.lru_cache(maxsize=None)
def _make(S: int, L: int, V: int):
    LP = -(-L // 8) * 8
    SPW = S // NW
    G = SPW // 2
    W = 2 * LP
    GQ = G // NQ
    assert S % (2 * NW) == 0 and W <= CH and GQ % NBUF == 0
    mesh = plsc.VectorSubcoreMesh(core_axis_name="c", subcore_axis_name="s")

    @functools.partial(
        pl.kernel,
        mesh=mesh,
        out_type=jax.ShapeDtypeStruct((S * LP, DP), jnp.float32),
        compiler_params=pltpu.CompilerParams(use_tc_tiling_on_sc=False),
        scratch_types=[
            pltpu.VMEM((GQ, CH), jnp.int32),
            *[pltpu.VMEM((W, DP), jnp.float32) for _ in range(NBUF)],
            *[pltpu.SemaphoreType.DMA for _ in range(2 * NBUF)],
        ],
    )
    def emb(x_hbm, lut_hbm, out_hbm, idx_v, *bs):
        bufs, gsem, ssem = bs[:NBUF], bs[NBUF:2 * NBUF], bs[2 * NBUF:]
        wid = lax.axis_index("s") * NC + lax.axis_index("c")
        base = wid * SPW * LP

        def start_gather(q, c, b):
            pltpu.async_copy(lut_hbm.at[idx_v.at[c, pl.ds(0, W)]], bufs[b],
                             gsem[b])

        def wait_gather(q, c, b):
            pltpu.make_async_copy(lut_hbm.at[idx_v.at[c, pl.ds(0, W)]],
                                  bufs[b], gsem[b]).wait()

        def start_scatter(q, c, b):
            pltpu.async_copy(
                bufs[b], out_hbm.at[pl.ds(base + (q * GQ + c) * W, W)],
                ssem[b])

        def wait_scatter(q, c, b):
            pltpu.make_async_copy(
                bufs[b], out_hbm.at[pl.ds(base + (q * GQ + c) * W, W)],
                ssem[b]).wait()

        for q in range(NQ):
            pltpu.sync_copy(x_hbm.at[wid, pl.ds(q * GQ, GQ)], idx_v)
            for c in range(AHEAD):
                start_gather(q, c, c)

            def body(i, carry, q=q):
                for b in range(NBUF):
                    c = i * NBUF + b
                    bn = (b + AHEAD) % NBUF
                    pl.when(c >= 2)(lambda: wait_scatter(q, c - 2, bn))
                    pl.when(c + AHEAD < GQ)(
                        lambda: start_gather(q, c + AHEAD, bn))
                    wait_gather(q, c, b)
                    start_scatter(q, c, b)
                return carry

            lax.fori_loop(0, GQ // NBUF, body, 0)

            for c in range(GQ - 2, GQ):
                wait_scatter(q, c, c % NBUF)

    return emb


def kernel(x, lut):
    S, L = x.shape
    V = lut.shape[0]
    LP = -(-L // 8) * 8
    lutp = jnp.pad(lut, ((0, 0), (0, DP - lut.shape[1]))) * SCALE
    x3 = jnp.pad(x.astype(jnp.int32).reshape(NW, S // NW, L),
                 ((0, 0), (0, 0), (0, LP - L)), mode="edge")
    x3 = x3.reshape(NW, S // NW // 2, 2 * LP)
    x3 = jnp.pad(x3, ((0, 0), (0, 0), (0, CH - 2 * LP)), mode="edge")
    out = _make(S, L, V)(x3, lutp)
    return out.reshape(S, LP, DP)[:, :L, :D]
